# baseline (speedup 1.0000x reference)
import functools

import jax
import jax.numpy as jnp
from jax import lax
from jax.experimental import pallas as pl
from jax.experimental.pallas import tpu as pltpu
from jax.experimental.pallas import tpu_sc as plsc

VOCAB = 1000000
EMB = 64
B = 16384
L = 16
NC = 2
NS = 16
NW = NC * NS
BPW = B // NW
NG = BPW // L

_mesh = plsc.VectorSubcoreMesh(core_axis_name="c", subcore_axis_name="s")


@functools.partial(
    pl.kernel,
    mesh=_mesh,
    out_type=(
        jax.ShapeDtypeStruct((128, 128), jnp.float32),
        jax.ShapeDtypeStruct((128, 128), jnp.float32),
    ),
    scratch_types=[
        pltpu.VMEM((BPW,), jnp.int32),
        pltpu.VMEM((BPW,), jnp.int32),
        pltpu.VMEM((BPW,), jnp.int32),
        pltpu.VMEM((L, 128), jnp.float32),
        pltpu.VMEM((L, 128), jnp.float32),
        pltpu.VMEM((L, 128), jnp.float32),
        pltpu.VMEM((L, 128), jnp.float32),
        pltpu.VMEM((L, 128), jnp.float32),
        pltpu.VMEM((L, 128), jnp.float32),
        pltpu.VMEM((4, 128), jnp.float32),
        pltpu.VMEM((4, 128), jnp.float32),
        pltpu.SemaphoreType.DMA,
        pltpu.SemaphoreType.DMA,
    ],
)
def _sc_dots(tw_hbm, cw_hbm, ng_hbm, temb_hbm, cemb_hbm,
             pos_hbm, neg_hbm,
             tw_v, cw_v, ng_v,
             tgt0, tgt1, ctx0, ctx1, ngr0, ngr1,
             pd_v, nd_v, sem0, sem1):
    wid = lax.axis_index("s") * NC + lax.axis_index("c")
    pltpu.sync_copy(tw_hbm.at[wid], tw_v)
    pltpu.sync_copy(cw_hbm.at[wid], cw_v)
    pltpu.sync_copy(ng_hbm.at[wid], ng_v)

    tgt_b = (tgt0, tgt1)
    ctx_b = (ctx0, ctx1)
    ngr_b = (ngr0, ngr1)
    sems = (sem0, sem1)

    def fire(g, par):
        ivt = tw_v[pl.ds(g * L, L)]
        ivc = cw_v[pl.ds(g * L, L)]
        ivn = ng_v[pl.ds(g * L, L)]
        for k in range(L):
            it = ivt[k]
            ic = ivc[k]
            iq = ivn[k]
            pltpu.async_copy(
                temb_hbm.at[it >> 3, pl.ds(((it >> 1) & 3) * 128, 128)],
                tgt_b[par].at[k], sems[par])
            pltpu.async_copy(
                cemb_hbm.at[ic >> 3, pl.ds(((ic >> 1) & 3) * 128, 128)],
                ctx_b[par].at[k], sems[par])
            pltpu.async_copy(
                temb_hbm.at[iq >> 3, pl.ds(((iq >> 1) & 3) * 128, 128)],
                ngr_b[par].at[k], sems[par])

    def drain(par):
        dummy = pos_hbm.at[pl.ds(0, L)]
        pltpu.make_async_copy(dummy, tgt_b[par], sems[par]).wait()
        pltpu.make_async_copy(dummy, ctx_b[par], sems[par]).wait()
        pltpu.make_async_copy(dummy, ngr_b[par], sems[par]).wait()

    def compute(g, par):
        cb, tb, nb = ctx_b[par], tgt_b[par], ngr_b[par]
        ivt = tw_v[pl.ds(g * L, L)]
        ivc = cw_v[pl.ds(g * L, L)]
        ivn = ng_v[pl.ds(g * L, L)]
        lane = lax.iota(jnp.int32, L)
        perms = [lane ^ sh for sh in (1, 2, 4, 8)]
        dnums = lax.GatherDimensionNumbers(
            offset_dims=(), collapsed_slice_dims=(0,), start_index_map=(0,))

        def lane_sum(v):
            for p in perms:
                v = v + lax.gather(
                    v, p[:, None], dnums, slice_sizes=(1,),
                    mode=lax.GatherScatterMode.PROMISE_IN_BOUNDS)
            return v

        acc_p = jnp.zeros((L,), jnp.float32)
        acc_n = jnp.zeros((L,), jnp.float32)
        for k in range(L):
            ot = (ivt[k] & 1) * EMB
            oc = (ivc[k] & 1) * EMB
            on = (ivn[k] & 1) * EMB
            c0 = cb[k, pl.ds(oc, L)]
            c1 = cb[k, pl.ds(oc + L, L)]
            c2 = cb[k, pl.ds(oc + 2 * L, L)]
            c3 = cb[k, pl.ds(oc + 3 * L, L)]
            pp = tb[k, pl.ds(ot, L)] * c0
            pp = pp + tb[k, pl.ds(ot + L, L)] * c1
            pp = pp + tb[k, pl.ds(ot + 2 * L, L)] * c2
            pp = pp + tb[k, pl.ds(ot + 3 * L, L)] * c3
            nn = nb[k, pl.ds(on, L)] * c0
            nn = nn + nb[k, pl.ds(on + L, L)] * c1
            nn = nn + nb[k, pl.ds(on + 2 * L, L)] * c2
            nn = nn + nb[k, pl.ds(on + 3 * L, L)] * c3
            acc_p = jnp.where(lane == k, lane_sum(pp), acc_p)
            acc_n = jnp.where(lane == k, lane_sum(nn), acc_n)
        pd_v[g >> 3, pl.ds((g & 7) * L, L)] = acc_p
        nd_v[g >> 3, pl.ds((g & 7) * L, L)] = acc_n

    fire(0, 0)

    def step(s, carry):
        g0 = 2 * s
        fire(g0 + 1, 1)
        drain(0)
        compute(g0, 0)

        @pl.when(s < NG // 2 - 1)
        def _():
            fire(g0 + 2, 0)

        drain(1)
        compute(g0 + 1, 1)
        return carry

    lax.fori_loop(0, NG // 2, step, 0)

    pltpu.sync_copy(pd_v, pos_hbm.at[pl.ds(wid * 4, 4)])
    pltpu.sync_copy(nd_v, neg_hbm.at[pl.ds(wid * 4, 4)])


def _loss_body(pos_ref, neg_ref, out_ref):
    p = pos_ref[...]
    n = -neg_ref[...]
    lp = jnp.minimum(p, 0.0) - jnp.log(1.0 + jnp.exp(-jnp.abs(p)))
    ln = jnp.minimum(n, 0.0) - jnp.log(1.0 + jnp.exp(-jnp.abs(n)))
    out_ref[0] = -(jnp.sum(lp) + jnp.sum(ln))


_loss = pl.pallas_call(
    _loss_body,
    out_shape=jax.ShapeDtypeStruct((1,), jnp.float32),
    in_specs=[
        pl.BlockSpec(memory_space=pltpu.VMEM),
        pl.BlockSpec(memory_space=pltpu.VMEM),
    ],
    out_specs=pl.BlockSpec(memory_space=pltpu.SMEM),
)


def kernel(target_word, context_word, negative_example, target_emb, context_emb):
    tw = target_word.astype(jnp.int32).reshape(NW, BPW)
    cw = context_word.astype(jnp.int32).reshape(NW, BPW)
    ng = negative_example.astype(jnp.int32).reshape(NW, BPW)
    t3 = target_emb.reshape(VOCAB // 8, 8 * EMB)
    c3 = context_emb.reshape(VOCAB // 8, 8 * EMB)
    pos, neg = _sc_dots(tw, cw, ng, t3, c3)
    loss = _loss(pos, neg)
    return loss[0]


# speedup vs baseline: 2.4603x; 2.4603x over previous
import functools

import jax
import jax.numpy as jnp
from jax import lax
from jax.experimental import pallas as pl
from jax.experimental.pallas import tpu as pltpu
from jax.experimental.pallas import tpu_sc as plsc

VOCAB = 1000000
EMB = 64
B = 16384
L = 16
NC = 2
NS = 16
NW = NC * NS
BPW = B // NW
NG = BPW // L

_mesh = plsc.VectorSubcoreMesh(core_axis_name="c", subcore_axis_name="s")


@functools.partial(
    pl.kernel,
    mesh=_mesh,
    out_type=(
        jax.ShapeDtypeStruct((128, 128), jnp.float32),
        jax.ShapeDtypeStruct((128, 128), jnp.float32),
    ),
    scratch_types=[
        pltpu.VMEM((BPW,), jnp.int32),
        pltpu.VMEM((BPW,), jnp.int32),
        pltpu.VMEM((BPW,), jnp.int32),
        pltpu.VMEM((8, 128), jnp.float32),
        pltpu.VMEM((8, 128), jnp.float32),
        pltpu.VMEM((8, 128), jnp.float32),
        pltpu.VMEM((8, 128), jnp.float32),
        pltpu.VMEM((8, 128), jnp.float32),
        pltpu.VMEM((8, 128), jnp.float32),
        pltpu.VMEM((4, 128), jnp.float32),
        pltpu.VMEM((4, 128), jnp.float32),
        pltpu.SemaphoreType.DMA,
        pltpu.SemaphoreType.DMA,
    ],
)
def _sc_dots(tw_hbm, cw_hbm, ng_hbm, temb_hbm, cemb_hbm,
             pos_hbm, neg_hbm,
             tw_v, cw_v, ng_v,
             tgt0, tgt1, ctx0, ctx1, ngr0, ngr1,
             pd_v, nd_v, sem0, sem1):
    wid = lax.axis_index("s") * NC + lax.axis_index("c")
    pltpu.sync_copy(tw_hbm.at[wid], tw_v)
    pltpu.sync_copy(cw_hbm.at[wid], cw_v)
    pltpu.sync_copy(ng_hbm.at[wid], ng_v)

    tgt_b = (tgt0, tgt1)
    ctx_b = (ctx0, ctx1)
    ngr_b = (ngr0, ngr1)
    sems = (sem0, sem1)

    def fire(g, par):
        ivt = tw_v[pl.ds(g * L, L)]
        ivc = cw_v[pl.ds(g * L, L)]
        ivn = ng_v[pl.ds(g * L, L)]
        for k in range(L):
            it = ivt[k]
            ic = ivc[k]
            iq = ivn[k]
            row, col = k // 2, (k % 2) * EMB
            pltpu.async_copy(temb_hbm.at[it >> 3, it & 7],
                             tgt_b[par].at[row, pl.ds(col, EMB)], sems[par])
            pltpu.async_copy(cemb_hbm.at[ic >> 3, ic & 7],
                             ctx_b[par].at[row, pl.ds(col, EMB)], sems[par])
            pltpu.async_copy(temb_hbm.at[iq >> 3, iq & 7],
                             ngr_b[par].at[row, pl.ds(col, EMB)], sems[par])

    def drain(par):
        dummy = pos_hbm.at[pl.ds(0, 8)]
        pltpu.make_async_copy(dummy, tgt_b[par], sems[par]).wait()
        pltpu.make_async_copy(dummy, ctx_b[par], sems[par]).wait()
        pltpu.make_async_copy(dummy, ngr_b[par], sems[par]).wait()

    def compute(g, par):
        cb, tb, nb = ctx_b[par], tgt_b[par], ngr_b[par]
        lane = lax.iota(jnp.int32, L)
        perms = [lane ^ sh for sh in (1, 2, 4, 8)]
        dnums = lax.GatherDimensionNumbers(
            offset_dims=(), collapsed_slice_dims=(0,), start_index_map=(0,))

        def lane_sum(v):
            for p in perms:
                v = v + lax.gather(
                    v, p[:, None], dnums, slice_sizes=(1,),
                    mode=lax.GatherScatterMode.PROMISE_IN_BOUNDS)
            return v

        acc_p = jnp.zeros((L,), jnp.float32)
        acc_n = jnp.zeros((L,), jnp.float32)
        for k in range(L):
            row, col = k // 2, (k % 2) * EMB
            c0 = cb[row, pl.ds(col, L)]
            c1 = cb[row, pl.ds(col + L, L)]
            c2 = cb[row, pl.ds(col + 2 * L, L)]
            c3 = cb[row, pl.ds(col + 3 * L, L)]
            pp = tb[row, pl.ds(col, L)] * c0
            pp = pp + tb[row, pl.ds(col + L, L)] * c1
            pp = pp + tb[row, pl.ds(col + 2 * L, L)] * c2
            pp = pp + tb[row, pl.ds(col + 3 * L, L)] * c3
            nn = nb[row, pl.ds(col, L)] * c0
            nn = nn + nb[row, pl.ds(col + L, L)] * c1
            nn = nn + nb[row, pl.ds(col + 2 * L, L)] * c2
            nn = nn + nb[row, pl.ds(col + 3 * L, L)] * c3
            acc_p = jnp.where(lane == k, lane_sum(pp), acc_p)
            acc_n = jnp.where(lane == k, lane_sum(nn), acc_n)
        pd_v[g >> 3, pl.ds((g & 7) * L, L)] = acc_p
        nd_v[g >> 3, pl.ds((g & 7) * L, L)] = acc_n

    fire(0, 0)

    def step(s, carry):
        g0 = 2 * s
        fire(g0 + 1, 1)
        drain(0)
        compute(g0, 0)

        @pl.when(s < NG // 2 - 1)
        def _():
            fire(g0 + 2, 0)

        drain(1)
        compute(g0 + 1, 1)
        return carry

    lax.fori_loop(0, NG // 2, step, 0)

    pltpu.sync_copy(pd_v, pos_hbm.at[pl.ds(wid * 4, 4)])
    pltpu.sync_copy(nd_v, neg_hbm.at[pl.ds(wid * 4, 4)])


def _loss_body(pos_ref, neg_ref, out_ref):
    p = pos_ref[...]
    n = -neg_ref[...]
    lp = jnp.minimum(p, 0.0) - jnp.log(1.0 + jnp.exp(-jnp.abs(p)))
    ln = jnp.minimum(n, 0.0) - jnp.log(1.0 + jnp.exp(-jnp.abs(n)))
    out_ref[0] = -(jnp.sum(lp) + jnp.sum(ln))


_loss = pl.pallas_call(
    _loss_body,
    out_shape=jax.ShapeDtypeStruct((1,), jnp.float32),
    in_specs=[
        pl.BlockSpec(memory_space=pltpu.VMEM),
        pl.BlockSpec(memory_space=pltpu.VMEM),
    ],
    out_specs=pl.BlockSpec(memory_space=pltpu.SMEM),
)


def kernel(target_word, context_word, negative_example, target_emb, context_emb):
    tw = target_word.astype(jnp.int32).reshape(NW, BPW)
    cw = context_word.astype(jnp.int32).reshape(NW, BPW)
    ng = negative_example.astype(jnp.int32).reshape(NW, BPW)
    t3 = target_emb.reshape(VOCAB // 8, 8, EMB)
    c3 = context_emb.reshape(VOCAB // 8, 8, EMB)
    pos, neg = _sc_dots(tw, cw, ng, t3, c3)
    loss = _loss(pos, neg)
    return loss[0]
